# baseline (speedup 1.0000x reference)
import functools

import jax
import jax.numpy as jnp
from jax import lax
from jax.experimental import pallas as pl
from jax.experimental.pallas import tpu as pltpu
from jax.experimental.pallas import tpu_sc as plsc

NC = 2
NS = 16
NW = NC * NS

CHUNK = 32
NBUF = 3


@functools.partial(jax.jit, static_argnums=(2, 3))
def _gather_rows(table, idx, n_idx, dim):
    b_per_w = n_idx // NW
    n_chunks = b_per_w // CHUNK
    n_groups = n_chunks // NBUF
    n_tail = n_chunks - n_groups * NBUF
    idx3 = idx.reshape(NW, n_chunks, CHUNK)

    mesh = plsc.VectorSubcoreMesh(core_axis_name="c", subcore_axis_name="s")

    @functools.partial(
        pl.kernel,
        mesh=mesh,
        out_type=jax.ShapeDtypeStruct((n_idx, dim), jnp.float32),
        scratch_types=[
            pltpu.VMEM((n_chunks, CHUNK), jnp.int32),
            pltpu.VMEM((NBUF, CHUNK, dim), jnp.float32),
            pltpu.SemaphoreType.DMA((NBUF,)),
            pltpu.SemaphoreType.DMA((NBUF,)),
        ],
    )
    def k(table_hbm, idx_hbm, out_hbm, idx_v, bufs, gsem, osem):
        wid = lax.axis_index("s") * NC + lax.axis_index("c")
        base = wid * b_per_w
        pltpu.sync_copy(idx_hbm.at[wid], idx_v)

        def wait_write(b):
            pltpu.make_async_copy(
                bufs.at[b], out_hbm.at[pl.ds(0, CHUNK)], osem.at[b]
            ).wait()

        def gather(c, b):
            return pltpu.async_copy(
                table_hbm.at[idx_v.at[c]], bufs.at[b], gsem.at[b]
            )

        def write(c, b):
            pltpu.async_copy(
                bufs.at[b], out_hbm.at[pl.ds(base + c * CHUNK, CHUNK)],
                osem.at[b],
            )

        def body(g, carry):
            gathers = []
            for b in range(NBUF):
                @pl.when(g > 0)
                def _(b=b):
                    wait_write(b)

                gathers.append(gather(g * NBUF + b, b))
            for b in range(NBUF):
                gathers[b].wait()
                write(g * NBUF + b, b)
            return carry

        lax.fori_loop(0, n_groups, body, 0)

        tail0 = n_groups * NBUF
        tail_gathers = []
        for t in range(n_tail):
            wait_write(t)
            tail_gathers.append(gather(tail0 + t, t))
        for t in range(n_tail):
            tail_gathers[t].wait()
            write(tail0 + t, t)

        for b in range(NBUF):
            wait_write(b)

    return k(table, idx3)


def kernel(position_ids, pe):
    batch, seq_len = position_ids.shape
    n_idx = batch * seq_len
    dim = pe.shape[1]
    flat = position_ids.reshape(n_idx)
    out = _gather_rows(pe, flat, n_idx, dim)
    return out.reshape(batch, seq_len, dim)


# speedup vs baseline: 1.0151x; 1.0151x over previous
import functools

import jax
import jax.numpy as jnp
from jax import lax
from jax.experimental import pallas as pl
from jax.experimental.pallas import tpu as pltpu
from jax.experimental.pallas import tpu_sc as plsc

NC = 2
NS = 16
NW = NC * NS

CHUNK = 16
NBUF = 4


@functools.partial(jax.jit, static_argnums=(2, 3))
def _gather_rows(table, idx, n_idx, dim):
    b_per_w = n_idx // NW
    n_chunks = b_per_w // CHUNK
    n_groups = n_chunks // NBUF
    idx3 = idx.reshape(NW, n_chunks, CHUNK)

    mesh = plsc.VectorSubcoreMesh(core_axis_name="c", subcore_axis_name="s")

    @functools.partial(
        pl.kernel,
        mesh=mesh,
        out_type=jax.ShapeDtypeStruct((n_idx, dim), jnp.float32),
        scratch_types=[
            pltpu.VMEM((n_chunks, CHUNK), jnp.int32),
            pltpu.VMEM((NBUF, CHUNK, dim), jnp.float32),
            pltpu.SemaphoreType.DMA((NBUF,)),
            pltpu.SemaphoreType.DMA((NBUF,)),
        ],
    )
    def k(table_hbm, idx_hbm, out_hbm, idx_v, bufs, gsem, osem):
        wid = lax.axis_index("s") * NC + lax.axis_index("c")
        base = wid * b_per_w
        pltpu.sync_copy(idx_hbm.at[wid], idx_v)

        def body(g, carry):
            gathers = []
            for b in range(NBUF):
                c = g * NBUF + b

                @pl.when(g > 0)
                def _(b=b):
                    pltpu.make_async_copy(
                        bufs.at[b], out_hbm.at[pl.ds(0, CHUNK)], osem.at[b]
                    ).wait()

                gathers.append(
                    pltpu.async_copy(
                        table_hbm.at[idx_v.at[c]], bufs.at[b], gsem.at[b]
                    )
                )
            for b in range(NBUF):
                c = g * NBUF + b
                gathers[b].wait()
                pltpu.async_copy(
                    bufs.at[b],
                    out_hbm.at[pl.ds(base + c * CHUNK, CHUNK)],
                    osem.at[b],
                )
            return carry

        lax.fori_loop(0, n_groups, body, 0)
        for b in range(NBUF):
            pltpu.make_async_copy(
                bufs.at[b], out_hbm.at[pl.ds(0, CHUNK)], osem.at[b]
            ).wait()

    return k(table, idx3)


def kernel(position_ids, pe):
    batch, seq_len = position_ids.shape
    n_idx = batch * seq_len
    dim = pe.shape[1]
    flat = position_ids.reshape(n_idx)
    out = _gather_rows(pe, flat, n_idx, dim)
    return out.reshape(batch, seq_len, dim)
